# baseline (speedup 1.0000x reference)
import functools

import jax
import jax.numpy as jnp
from jax import lax
from jax.experimental import pallas as pl
from jax.experimental.pallas import tpu as pltpu
from jax.experimental.pallas import tpu_sc as plsc

_N = 10000
_E = 320000
_DF = 128
_DE = 16
_H = 64
_W = 128
_F32 = jnp.float32

_NC = 2
_NS = 16
_NW = _NC * _NS
_EPW = _E // _NW
_CH = 80
_NCHUNK = _EPW // _CH
_RPS = 624
_RTAIL = _N - _NS * _RPS
_DW = 16

_mesh = plsc.VectorSubcoreMesh(core_axis_name="c", subcore_axis_name="s")


def _zero_acc(zeros_hbm, acc_sh, s):
    srow = pl.multiple_of(s * _RPS, 8)
    pltpu.sync_copy(zeros_hbm.at[pl.ds(srow, _RPS)],
                    acc_sh.at[pl.ds(srow, _RPS)])

    @pl.when(s == _NS - 1)
    def _zero_tail():
        pltpu.sync_copy(zeros_hbm.at[pl.ds(_NS * _RPS, _RTAIL)],
                        acc_sh.at[pl.ds(_NS * _RPS, _RTAIL)])


def _acc_out(acc_sh, out_hbm, c, s):
    srow = pl.multiple_of(s * _RPS, 8)
    orow = pl.multiple_of(c * _N + s * _RPS, 8)
    pltpu.sync_copy(acc_sh.at[pl.ds(srow, _RPS)],
                    out_hbm.at[pl.ds(orow, _RPS)])

    @pl.when(s == _NS - 1)
    def _out_tail():
        pltpu.sync_copy(acc_sh.at[pl.ds(_NS * _RPS, _RTAIL)],
                        out_hbm.at[pl.ds(c * _N + _NS * _RPS, _RTAIL)])


@functools.partial(
    pl.kernel,
    out_type=jax.ShapeDtypeStruct((_NC * _N, _DW), _F32),
    mesh=_mesh,
    compiler_params=pltpu.CompilerParams(use_tc_tiling_on_sc=False),
    scratch_types=[
        pltpu.VMEM_SHARED((_N, _DW), _F32),
        pltpu.VMEM((_NCHUNK, _CH), jnp.int32),
        pltpu.VMEM((_CH, _DW), _F32),
    ],
)
def _sc_deg(col2d_hbm, ones_hbm, zeros_hbm, out_hbm, acc_sh, cidx_v, ones_v):
    c = lax.axis_index("c")
    s = lax.axis_index("s")
    wid = s * _NC + c
    _zero_acc(zeros_hbm, acc_sh, s)
    pltpu.sync_copy(col2d_hbm.at[wid], cidx_v)
    pltpu.sync_copy(ones_hbm, ones_v)
    plsc.subcore_barrier()

    def body(j, carry):
        pltpu.sync_copy(ones_v, acc_sh.at[cidx_v.at[j]], add=True)
        return carry

    lax.fori_loop(0, _NCHUNK, body, 0)
    plsc.subcore_barrier()
    _acc_out(acc_sh, out_hbm, c, s)


@functools.partial(
    pl.kernel,
    out_type=jax.ShapeDtypeStruct((_NC * _N, _H), _F32),
    mesh=_mesh,
    compiler_params=pltpu.CompilerParams(use_tc_tiling_on_sc=False),
    scratch_types=[
        pltpu.VMEM_SHARED((_N, _H), _F32),
        pltpu.VMEM((_NCHUNK, _CH), jnp.int32),
        pltpu.VMEM((_NCHUNK, _CH), jnp.int32),
        pltpu.VMEM((_CH, _H), _F32),
        pltpu.VMEM((_CH, _H), _F32),
        pltpu.SemaphoreType.DMA,
        pltpu.SemaphoreType.DMA,
    ],
)
def _sc_scatter(hs_hbm, row2d_hbm, col2d_hbm, zeros_hbm, out_hbm,
                acc_sh, ridx_v, cidx_v, rows0_v, rows1_v, gsem0, gsem1):
    c = lax.axis_index("c")
    s = lax.axis_index("s")
    wid = s * _NC + c
    _zero_acc(zeros_hbm, acc_sh, s)
    pltpu.sync_copy(row2d_hbm.at[wid], ridx_v)
    pltpu.sync_copy(col2d_hbm.at[wid], cidx_v)
    plsc.subcore_barrier()

    def gstart(j, buf, sem):
        pltpu.async_copy(hs_hbm.at[ridx_v.at[j]], buf, sem)

    def gwait(buf, sem):
        pltpu.make_async_copy(hs_hbm.at[ridx_v.at[0]], buf, sem).wait()

    def scat(j, buf):
        pltpu.sync_copy(buf, acc_sh.at[cidx_v.at[j]], add=True)

    gstart(0, rows0_v, gsem0)

    def body(k, carry):
        j0 = 2 * k
        gstart(j0 + 1, rows1_v, gsem1)
        gwait(rows0_v, gsem0)
        scat(j0, rows0_v)
        gstart(j0 + 2, rows0_v, gsem0)
        gwait(rows1_v, gsem1)
        scat(j0 + 1, rows1_v)
        return carry

    lax.fori_loop(0, (_NCHUNK - 1) // 2, body, 0)
    gwait(rows0_v, gsem0)
    scat(_NCHUNK - 1, rows0_v)
    plsc.subcore_barrier()
    _acc_out(acc_sh, out_hbm, c, s)


@functools.partial(
    pl.kernel,
    out_type=jax.ShapeDtypeStruct((_E, _W), jnp.bfloat16),
    mesh=_mesh,
    compiler_params=pltpu.CompilerParams(use_tc_tiling_on_sc=False),
    scratch_types=[
        pltpu.VMEM((_NCHUNK, _CH), jnp.int32),
        pltpu.VMEM((_NCHUNK, _CH), jnp.int32),
        pltpu.VMEM((_CH, _H), jnp.bfloat16),
        pltpu.VMEM((_CH, _H), jnp.bfloat16),
        pltpu.VMEM((_CH, _H), jnp.bfloat16),
        pltpu.VMEM((_CH, _H), jnp.bfloat16),
        pltpu.SemaphoreType.DMA,
        pltpu.SemaphoreType.DMA,
        pltpu.SemaphoreType.DMA,
        pltpu.SemaphoreType.DMA,
        pltpu.SemaphoreType.DMA,
        pltpu.SemaphoreType.DMA,
        pltpu.SemaphoreType.DMA,
        pltpu.SemaphoreType.DMA,
    ],
)
def _sc_gather2(p_hbm, q_hbm, row2d_hbm, col2d_hbm, t_hbm,
                ridx_v, cidx_v, p0_v, q0_v, p1_v, q1_v,
                sp0, sq0, sp1, sq1, wp0, wq0, wp1, wq1):
    c = lax.axis_index("c")
    s = lax.axis_index("s")
    wid = s * _NC + c
    base = wid * _EPW
    pltpu.sync_copy(row2d_hbm.at[wid], ridx_v)
    pltpu.sync_copy(col2d_hbm.at[wid], cidx_v)

    def gstart(j, tab, idx, buf, sem):
        pltpu.async_copy(tab.at[idx.at[j]], buf, sem)

    def gwait(buf, sem):
        pltpu.make_async_copy(p_hbm.at[ridx_v.at[0]], buf, sem).wait()

    def wstart(j, buf, colo, sem):
        off = pl.multiple_of(base + j * _CH, 8)
        pltpu.async_copy(buf, t_hbm.at[pl.ds(off, _CH), pl.ds(colo, _H)], sem)

    def wwait(buf, colo, sem):
        pltpu.make_async_copy(
            buf, t_hbm.at[pl.ds(0, _CH), pl.ds(colo, _H)], sem).wait()

    gstart(0, p_hbm, ridx_v, p0_v, sp0)
    gstart(0, q_hbm, cidx_v, q0_v, sq0)

    def body(k, carry):
        j0 = 2 * k
        gstart(j0 + 1, p_hbm, ridx_v, p1_v, sp1)
        gstart(j0 + 1, q_hbm, cidx_v, q1_v, sq1)
        gwait(p0_v, sp0)
        wstart(j0, p0_v, 0, wp0)
        gwait(q0_v, sq0)
        wstart(j0, q0_v, _H, wq0)
        wwait(p0_v, 0, wp0)
        wwait(q0_v, _H, wq0)
        gstart(j0 + 2, p_hbm, ridx_v, p0_v, sp0)
        gstart(j0 + 2, q_hbm, cidx_v, q0_v, sq0)
        gwait(p1_v, sp1)
        wstart(j0 + 1, p1_v, 0, wp1)
        gwait(q1_v, sq1)
        wstart(j0 + 1, q1_v, _H, wq1)
        wwait(p1_v, 0, wp1)
        wwait(q1_v, _H, wq1)
        return carry

    lax.fori_loop(0, (_NCHUNK - 1) // 2, body, 0)
    gwait(p0_v, sp0)
    wstart(_NCHUNK - 1, p0_v, 0, wp0)
    gwait(q0_v, sq0)
    wstart(_NCHUNK - 1, q0_v, _H, wq0)
    wwait(p0_v, 0, wp0)
    wwait(q0_v, _H, wq0)


def _tc_prep_body(degp_ref, x_ref, w_ref, hs_ref, dinv_ref):
    deg = degp_ref[0:_N, 0:1] + degp_ref[_N:2 * _N, 0:1] + 1.0
    dinv = lax.rsqrt(deg)
    hp = jnp.dot(x_ref[...], w_ref[...], preferred_element_type=_F32)
    hs_ref[...] = hp * dinv
    dinv_ref[...] = dinv


def _tc_glue_body(part_ref, hs_ref, dinv_ref, b_ref, w_ref, out_ref):
    s_sum = part_ref[0:_N, :] + part_ref[_N:2 * _N, :]
    h = jnp.maximum(
        dinv_ref[...] * (s_sum + hs_ref[...]) + b_ref[...], 0.0)
    hp = jnp.dot(h, w_ref[...], preferred_element_type=_F32)
    out_ref[...] = dinv_ref[...] * hp


def _tc_fin_body(part_ref, hs_ref, dinv_ref, b_ref, wa_ref, wb_ref,
                 p_ref, q_ref):
    s_sum = part_ref[0:_N, :] + part_ref[_N:2 * _N, :]
    h = jnp.maximum(
        dinv_ref[...] * (s_sum + hs_ref[...]) + b_ref[...], 0.0)
    p_ref[...] = jnp.dot(
        h, wa_ref[...], preferred_element_type=_F32).astype(jnp.bfloat16)
    q_ref[...] = jnp.dot(
        h, wb_ref[...], preferred_element_type=_F32).astype(jnp.bfloat16)


_BE = 12800


def _tc_edge_body(t_ref, ea_ref, wc_ref, b1_ref, w2_ref, b2_ref,
                  w3t_ref, b3t_ref, out_ref):
    tf = t_ref[...].astype(_F32)
    o = (tf[:, 0:_H] + tf[:, _H:_W]
         + jnp.dot(ea_ref[...], wc_ref[...], preferred_element_type=_F32)
         + b1_ref[...])
    o = jnp.maximum(o, 0.0)
    o = jnp.maximum(
        jnp.dot(o, w2_ref[...], preferred_element_type=_F32) + b2_ref[...], 0.0)
    ot = jax.lax.dot_general(w3t_ref[...], o, (((1,), (1,)), ((), ())),
                             preferred_element_type=_F32)
    out_ref[...] = ot + b3t_ref[...]


def _full(shape):
    return pl.BlockSpec(shape, lambda i: tuple(0 for _ in shape))


def kernel(x, edge_index, edge_attr, W1, b1, W2, b2, W3, b3,
           Wm1, bm1, Wm2, bm2, Wm3, bm3):
    ei = edge_index.astype(jnp.int32)
    row2d = ei[0].reshape(_NW, _NCHUNK, _CH)
    col2d = ei[1].reshape(_NW, _NCHUNK, _CH)
    ones16 = jnp.ones((_CH, _DW), _F32)
    zeros16 = jnp.zeros((_N, _DW), _F32)
    zeros64 = jnp.zeros((_N, _H), _F32)

    degp = _sc_deg(col2d, ones16, zeros16)

    hs1, dinv = pl.pallas_call(
        _tc_prep_body,
        out_shape=(jax.ShapeDtypeStruct((_N, _H), _F32),
                   jax.ShapeDtypeStruct((_N, 1), _F32)),
    )(degp, x, W1)

    part1 = _sc_scatter(hs1, row2d, col2d, zeros64)

    glue = pl.pallas_call(
        _tc_glue_body,
        out_shape=jax.ShapeDtypeStruct((_N, _H), _F32),
    )
    hs2 = glue(part1, hs1, dinv, b1.reshape(1, _H), W2)
    part2 = _sc_scatter(hs2, row2d, col2d, zeros64)
    hs3 = glue(part2, hs2, dinv, b2.reshape(1, _H), W3)
    part3 = _sc_scatter(hs3, row2d, col2d, zeros64)

    p, q = pl.pallas_call(
        _tc_fin_body,
        out_shape=(jax.ShapeDtypeStruct((_N, _H), jnp.bfloat16),
                   jax.ShapeDtypeStruct((_N, _H), jnp.bfloat16)),
    )(part3, hs3, dinv, b3.reshape(1, _H), Wm1[0:_H], Wm1[_H:2 * _H])

    t = _sc_gather2(p, q, row2d, col2d)

    out = pl.pallas_call(
        _tc_edge_body,
        grid=(_E // _BE,),
        in_specs=[
            pl.BlockSpec((_BE, _W), lambda i: (i, 0)),
            pl.BlockSpec((_BE, _DE), lambda i: (i, 0)),
            _full((_DE, _H)),
            _full((1, _H)),
            _full((_H, 32)),
            _full((1, 32)),
            _full((2, 32)),
            _full((2, 1)),
        ],
        out_specs=pl.BlockSpec((2, _BE), lambda i: (0, i)),
        out_shape=jax.ShapeDtypeStruct((2, _E), _F32),
    )(t, edge_attr, Wm1[2 * _H:], bm1.reshape(1, _H),
      Wm2, bm2.reshape(1, 32), Wm3.T, bm3.reshape(2, 1))
    return out.T


# speedup vs baseline: 1.4114x; 1.4114x over previous
import functools

import jax
import jax.numpy as jnp
from jax import lax
from jax.experimental import pallas as pl
from jax.experimental.pallas import tpu as pltpu
from jax.experimental.pallas import tpu_sc as plsc

_N = 10000
_E = 320000
_DF = 128
_DE = 16
_H = 64
_W = 128
_F32 = jnp.float32

_NC = 2
_NS = 16
_NW = _NC * _NS
_EPW = _E // _NW
_CH = 80
_NCHUNK = _EPW // _CH
_RPS = 624
_RTAIL = _N - _NS * _RPS
_DW = 16

_mesh = plsc.VectorSubcoreMesh(core_axis_name="c", subcore_axis_name="s")


def _zero_acc(zeros_hbm, acc_sh, s):
    srow = pl.multiple_of(s * _RPS, 8)
    pltpu.sync_copy(zeros_hbm.at[pl.ds(srow, _RPS)],
                    acc_sh.at[pl.ds(srow, _RPS)])

    @pl.when(s == _NS - 1)
    def _zero_tail():
        pltpu.sync_copy(zeros_hbm.at[pl.ds(_NS * _RPS, _RTAIL)],
                        acc_sh.at[pl.ds(_NS * _RPS, _RTAIL)])


def _acc_out(acc_sh, out_hbm, c, s):
    srow = pl.multiple_of(s * _RPS, 8)
    orow = pl.multiple_of(c * _N + s * _RPS, 8)
    pltpu.sync_copy(acc_sh.at[pl.ds(srow, _RPS)],
                    out_hbm.at[pl.ds(orow, _RPS)])

    @pl.when(s == _NS - 1)
    def _out_tail():
        pltpu.sync_copy(acc_sh.at[pl.ds(_NS * _RPS, _RTAIL)],
                        out_hbm.at[pl.ds(c * _N + _NS * _RPS, _RTAIL)])


@functools.partial(
    pl.kernel,
    out_type=jax.ShapeDtypeStruct((_NC * _N, _DW), _F32),
    mesh=_mesh,
    compiler_params=pltpu.CompilerParams(use_tc_tiling_on_sc=False),
    scratch_types=[
        pltpu.VMEM_SHARED((_N, _DW), _F32),
        pltpu.VMEM((_NCHUNK, _CH), jnp.int32),
        pltpu.VMEM((_CH, _DW), _F32),
    ],
)
def _sc_deg(col2d_hbm, ones_hbm, zeros_hbm, out_hbm, acc_sh, cidx_v, ones_v):
    c = lax.axis_index("c")
    s = lax.axis_index("s")
    wid = s * _NC + c
    _zero_acc(zeros_hbm, acc_sh, s)
    pltpu.sync_copy(col2d_hbm.at[wid], cidx_v)
    pltpu.sync_copy(ones_hbm, ones_v)
    plsc.subcore_barrier()

    def body(j, carry):
        pltpu.sync_copy(ones_v, acc_sh.at[cidx_v.at[j]], add=True)
        return carry

    lax.fori_loop(0, _NCHUNK, body, 0)
    plsc.subcore_barrier()
    _acc_out(acc_sh, out_hbm, c, s)


@functools.partial(
    pl.kernel,
    out_type=jax.ShapeDtypeStruct((_NC * _N, _H), _F32),
    mesh=_mesh,
    compiler_params=pltpu.CompilerParams(use_tc_tiling_on_sc=False),
    scratch_types=[
        pltpu.VMEM_SHARED((_N, _H), _F32),
        pltpu.VMEM((_NCHUNK, _CH), jnp.int32),
        pltpu.VMEM((_NCHUNK, _CH), jnp.int32),
        pltpu.VMEM((_CH, _H), _F32),
        pltpu.VMEM((_CH, _H), _F32),
        pltpu.SemaphoreType.DMA,
        pltpu.SemaphoreType.DMA,
    ],
)
def _sc_scatter(hs_hbm, row2d_hbm, col2d_hbm, zeros_hbm, out_hbm,
                acc_sh, ridx_v, cidx_v, rows0_v, rows1_v, gsem0, gsem1):
    c = lax.axis_index("c")
    s = lax.axis_index("s")
    wid = s * _NC + c
    _zero_acc(zeros_hbm, acc_sh, s)
    pltpu.sync_copy(row2d_hbm.at[wid], ridx_v)
    pltpu.sync_copy(col2d_hbm.at[wid], cidx_v)
    plsc.subcore_barrier()

    def gstart(j, buf, sem):
        pltpu.async_copy(hs_hbm.at[ridx_v.at[j]], buf, sem)

    def gwait(buf, sem):
        pltpu.make_async_copy(hs_hbm.at[ridx_v.at[0]], buf, sem).wait()

    def scat(j, buf):
        pltpu.sync_copy(buf, acc_sh.at[cidx_v.at[j]], add=True)

    gstart(0, rows0_v, gsem0)

    def body(k, carry):
        j0 = 2 * k
        gstart(j0 + 1, rows1_v, gsem1)
        gwait(rows0_v, gsem0)
        scat(j0, rows0_v)
        gstart(j0 + 2, rows0_v, gsem0)
        gwait(rows1_v, gsem1)
        scat(j0 + 1, rows1_v)
        return carry

    lax.fori_loop(0, (_NCHUNK - 1) // 2, body, 0)
    gwait(rows0_v, gsem0)
    scat(_NCHUNK - 1, rows0_v)
    plsc.subcore_barrier()
    _acc_out(acc_sh, out_hbm, c, s)


@functools.partial(
    pl.kernel,
    out_type=jax.ShapeDtypeStruct((_E, _W), _F32),
    mesh=_mesh,
    compiler_params=pltpu.CompilerParams(use_tc_tiling_on_sc=False),
    scratch_types=[
        pltpu.VMEM((_NCHUNK, _CH), jnp.int32),
        pltpu.VMEM((_NCHUNK, _CH), jnp.int32),
        pltpu.VMEM((_CH, _H), _F32),
        pltpu.VMEM((_CH, _H), _F32),
        pltpu.VMEM((_CH, _H), _F32),
        pltpu.VMEM((_CH, _H), _F32),
        pltpu.SemaphoreType.DMA,
        pltpu.SemaphoreType.DMA,
        pltpu.SemaphoreType.DMA,
        pltpu.SemaphoreType.DMA,
        pltpu.SemaphoreType.DMA,
        pltpu.SemaphoreType.DMA,
        pltpu.SemaphoreType.DMA,
        pltpu.SemaphoreType.DMA,
    ],
)
def _sc_gather2(p_hbm, q_hbm, row2d_hbm, col2d_hbm, t_hbm,
                ridx_v, cidx_v, p0_v, q0_v, p1_v, q1_v,
                sp0, sq0, sp1, sq1, wp0, wq0, wp1, wq1):
    c = lax.axis_index("c")
    s = lax.axis_index("s")
    wid = s * _NC + c
    base = wid * _EPW
    pltpu.sync_copy(row2d_hbm.at[wid], ridx_v)
    pltpu.sync_copy(col2d_hbm.at[wid], cidx_v)

    def gstart(j, tab, idx, buf, sem):
        pltpu.async_copy(tab.at[idx.at[j]], buf, sem)

    def gwait(buf, sem):
        pltpu.make_async_copy(p_hbm.at[ridx_v.at[0]], buf, sem).wait()

    def wstart(j, buf, colo, sem):
        off = pl.multiple_of(base + j * _CH, 8)
        pltpu.async_copy(buf, t_hbm.at[pl.ds(off, _CH), pl.ds(colo, _H)], sem)

    def wwait(buf, colo, sem):
        pltpu.make_async_copy(
            buf, t_hbm.at[pl.ds(0, _CH), pl.ds(colo, _H)], sem).wait()

    gstart(0, p_hbm, ridx_v, p0_v, sp0)
    gstart(0, q_hbm, cidx_v, q0_v, sq0)

    def body(k, carry):
        j0 = 2 * k
        gstart(j0 + 1, p_hbm, ridx_v, p1_v, sp1)
        gstart(j0 + 1, q_hbm, cidx_v, q1_v, sq1)
        gwait(p0_v, sp0)
        wstart(j0, p0_v, 0, wp0)
        gwait(q0_v, sq0)
        wstart(j0, q0_v, _H, wq0)
        wwait(p0_v, 0, wp0)
        wwait(q0_v, _H, wq0)
        gstart(j0 + 2, p_hbm, ridx_v, p0_v, sp0)
        gstart(j0 + 2, q_hbm, cidx_v, q0_v, sq0)
        gwait(p1_v, sp1)
        wstart(j0 + 1, p1_v, 0, wp1)
        gwait(q1_v, sq1)
        wstart(j0 + 1, q1_v, _H, wq1)
        wwait(p1_v, 0, wp1)
        wwait(q1_v, _H, wq1)
        return carry

    lax.fori_loop(0, (_NCHUNK - 1) // 2, body, 0)
    gwait(p0_v, sp0)
    wstart(_NCHUNK - 1, p0_v, 0, wp0)
    gwait(q0_v, sq0)
    wstart(_NCHUNK - 1, q0_v, _H, wq0)
    wwait(p0_v, 0, wp0)
    wwait(q0_v, _H, wq0)


def _tc_prep_body(degp_ref, x_ref, w_ref, hs_ref, dinv_ref):
    deg = degp_ref[0:_N, 0:1] + degp_ref[_N:2 * _N, 0:1] + 1.0
    dinv = lax.rsqrt(deg)
    hp = jnp.dot(x_ref[...], w_ref[...], preferred_element_type=_F32)
    hs_ref[...] = hp * dinv
    dinv_ref[...] = dinv


def _tc_glue_body(part_ref, hs_ref, dinv_ref, b_ref, w_ref, out_ref):
    s_sum = part_ref[0:_N, :] + part_ref[_N:2 * _N, :]
    h = jnp.maximum(
        dinv_ref[...] * (s_sum + hs_ref[...]) + b_ref[...], 0.0)
    hp = jnp.dot(h, w_ref[...], preferred_element_type=_F32)
    out_ref[...] = dinv_ref[...] * hp


def _tc_fin_body(part_ref, hs_ref, dinv_ref, b_ref, wa_ref, wb_ref,
                 p_ref, q_ref):
    s_sum = part_ref[0:_N, :] + part_ref[_N:2 * _N, :]
    h = jnp.maximum(
        dinv_ref[...] * (s_sum + hs_ref[...]) + b_ref[...], 0.0)
    p_ref[...] = jnp.dot(h, wa_ref[...], preferred_element_type=_F32)
    q_ref[...] = jnp.dot(h, wb_ref[...], preferred_element_type=_F32)


_BE = 12800


def _tc_edge_body(t_ref, ea_ref, wc_ref, b1_ref, w2_ref, b2_ref,
                  w3t_ref, b3t_ref, out_ref):
    tf = t_ref[...]
    o = (tf[:, 0:_H] + tf[:, _H:_W]
         + jnp.dot(ea_ref[...], wc_ref[...], preferred_element_type=_F32)
         + b1_ref[...])
    o = jnp.maximum(o, 0.0)
    o = jnp.maximum(
        jnp.dot(o, w2_ref[...], preferred_element_type=_F32) + b2_ref[...], 0.0)
    ot = jax.lax.dot_general(w3t_ref[...], o, (((1,), (1,)), ((), ())),
                             preferred_element_type=_F32)
    out_ref[...] = ot + b3t_ref[...]


def _full(shape):
    return pl.BlockSpec(shape, lambda i: tuple(0 for _ in shape))


def kernel(x, edge_index, edge_attr, W1, b1, W2, b2, W3, b3,
           Wm1, bm1, Wm2, bm2, Wm3, bm3):
    ei = edge_index.astype(jnp.int32)
    row2d = ei[0].reshape(_NW, _NCHUNK, _CH)
    col2d = ei[1].reshape(_NW, _NCHUNK, _CH)
    ones16 = jnp.ones((_CH, _DW), _F32)
    zeros16 = jnp.zeros((_N, _DW), _F32)
    zeros64 = jnp.zeros((_N, _H), _F32)

    degp = _sc_deg(col2d, ones16, zeros16)

    hs1, dinv = pl.pallas_call(
        _tc_prep_body,
        out_shape=(jax.ShapeDtypeStruct((_N, _H), _F32),
                   jax.ShapeDtypeStruct((_N, 1), _F32)),
    )(degp, x, W1)

    part1 = _sc_scatter(hs1, row2d, col2d, zeros64)

    glue = pl.pallas_call(
        _tc_glue_body,
        out_shape=jax.ShapeDtypeStruct((_N, _H), _F32),
    )
    hs2 = glue(part1, hs1, dinv, b1.reshape(1, _H), W2)
    part2 = _sc_scatter(hs2, row2d, col2d, zeros64)
    hs3 = glue(part2, hs2, dinv, b2.reshape(1, _H), W3)
    part3 = _sc_scatter(hs3, row2d, col2d, zeros64)

    p, q = pl.pallas_call(
        _tc_fin_body,
        out_shape=(jax.ShapeDtypeStruct((_N, _H), _F32),
                   jax.ShapeDtypeStruct((_N, _H), _F32)),
    )(part3, hs3, dinv, b3.reshape(1, _H), Wm1[0:_H], Wm1[_H:2 * _H])

    t = _sc_gather2(p, q, row2d, col2d)

    out = pl.pallas_call(
        _tc_edge_body,
        grid=(_E // _BE,),
        in_specs=[
            pl.BlockSpec((_BE, _W), lambda i: (i, 0)),
            pl.BlockSpec((_BE, _DE), lambda i: (i, 0)),
            _full((_DE, _H)),
            _full((1, _H)),
            _full((_H, 32)),
            _full((1, 32)),
            _full((2, 32)),
            _full((2, 1)),
        ],
        out_specs=pl.BlockSpec((2, _BE), lambda i: (0, i)),
        out_shape=jax.ShapeDtypeStruct((2, _E), _F32),
    )(t, edge_attr, Wm1[2 * _H:], bm1.reshape(1, _H),
      Wm2, bm2.reshape(1, 32), Wm3.T, bm3.reshape(2, 1))
    return out.T
